# Initial kernel scaffold; baseline (speedup 1.0000x reference)
import jax
import jax.numpy as jnp
from jax.experimental import pallas as pl


def kernel(x, W, b):
    raise NotImplementedError("write your pallas kernel here")


# speedup vs baseline: 1.4116x; 1.4116x over previous
import jax
import jax.numpy as jnp
from jax import lax
from jax.experimental import pallas as pl

_N_TOKENS = 32768
_D = 768
_E = 64
_TM = 512


def _router_body(x_ref, w_ref, b_ref, scores_ref, ts_ref, ti_ref):
    x = x_ref[...]
    w = w_ref[...]
    logits = lax.dot_general(
        x, w, (((1,), (1,)), ((), ())), preferred_element_type=jnp.float32
    )
    logits = logits + b_ref[...]
    m = jnp.max(logits, axis=-1, keepdims=True)
    e = jnp.exp(logits - m)
    s = jnp.sum(e, axis=-1, keepdims=True)
    scores = e * (1.0 / s)
    scores_ref[...] = scores

    iota = lax.broadcasted_iota(jnp.int32, scores.shape, 1)
    m1 = jnp.max(scores, axis=-1, keepdims=True)
    i1 = jnp.min(jnp.where(scores == m1, iota, _E), axis=-1, keepdims=True)
    masked = jnp.where(iota == i1, -jnp.inf, scores)
    m2 = jnp.max(masked, axis=-1, keepdims=True)
    i2 = jnp.min(jnp.where(masked == m2, iota, _E), axis=-1, keepdims=True)
    ts_ref[...] = jnp.concatenate([m1, m2], axis=-1)
    ti_ref[...] = jnp.concatenate([i1, i2], axis=-1)


def kernel(x, W, b):
    scores, ts, ti = pl.pallas_call(
        _router_body,
        grid=(_N_TOKENS // _TM,),
        in_specs=[
            pl.BlockSpec((_TM, _D), lambda i: (i, 0)),
            pl.BlockSpec((_E, _D), lambda i: (0, 0)),
            pl.BlockSpec((1, _E), lambda i: (0, 0)),
        ],
        out_specs=[
            pl.BlockSpec((_TM, _E), lambda i: (i, 0)),
            pl.BlockSpec((_TM, 2), lambda i: (i, 0)),
            pl.BlockSpec((_TM, 2), lambda i: (i, 0)),
        ],
        out_shape=[
            jax.ShapeDtypeStruct((_N_TOKENS, _E), jnp.float32),
            jax.ShapeDtypeStruct((_N_TOKENS, 2), jnp.float32),
            jax.ShapeDtypeStruct((_N_TOKENS, 2), jnp.int32),
        ],
    )(x, W, b.reshape(1, _E))
    return ts, ti, scores


# speedup vs baseline: 1.7668x; 1.2516x over previous
import jax
import jax.numpy as jnp
from jax import lax
from jax.experimental import pallas as pl

_N_TOKENS = 32768
_D = 768
_E = 64
_TM = 1024


def _router_body(x_ref, w_ref, b_ref, scores_ref, ts_ref, ti_ref):
    x = x_ref[...]
    w = w_ref[...]
    logits = lax.dot_general(
        x, w, (((1,), (1,)), ((), ())), preferred_element_type=jnp.float32
    )
    logits = logits + b_ref[...]
    m = jnp.max(logits, axis=-1, keepdims=True)
    e = jnp.exp(logits - m)
    s = jnp.sum(e, axis=-1, keepdims=True)
    scores = e * (1.0 / s)
    scores_ref[...] = scores

    iota = lax.broadcasted_iota(jnp.int32, scores.shape, 1)
    m1 = jnp.max(scores, axis=-1, keepdims=True)
    i1 = jnp.min(jnp.where(scores == m1, iota, _E), axis=-1, keepdims=True)
    masked = jnp.where(iota == i1, -jnp.inf, scores)
    m2 = jnp.max(masked, axis=-1, keepdims=True)
    i2 = jnp.min(jnp.where(masked == m2, iota, _E), axis=-1, keepdims=True)
    ts_ref[...] = jnp.concatenate([m1, m2], axis=-1)
    ti_ref[...] = jnp.concatenate([i1, i2], axis=-1)


def kernel(x, W, b):
    scores, ts, ti = pl.pallas_call(
        _router_body,
        grid=(_N_TOKENS // _TM,),
        in_specs=[
            pl.BlockSpec((_TM, _D), lambda i: (i, 0)),
            pl.BlockSpec((_E, _D), lambda i: (0, 0)),
            pl.BlockSpec((1, _E), lambda i: (0, 0)),
        ],
        out_specs=[
            pl.BlockSpec((_TM, _E), lambda i: (i, 0)),
            pl.BlockSpec((_TM, 2), lambda i: (i, 0)),
            pl.BlockSpec((_TM, 2), lambda i: (i, 0)),
        ],
        out_shape=[
            jax.ShapeDtypeStruct((_N_TOKENS, _E), jnp.float32),
            jax.ShapeDtypeStruct((_N_TOKENS, 2), jnp.float32),
            jax.ShapeDtypeStruct((_N_TOKENS, 2), jnp.int32),
        ],
    )(x, W, b.reshape(1, _E))
    return ts, ti, scores


# speedup vs baseline: 2.0036x; 1.1340x over previous
import jax
import jax.numpy as jnp
from jax import lax
from jax.experimental import pallas as pl

_N_TOKENS = 32768
_D = 768
_E = 64
_TM = 2048


def _router_body(x_ref, w_ref, b_ref, scores_ref, ts_ref, ti_ref):
    x = x_ref[...]
    w = w_ref[...]
    logits = lax.dot_general(
        x, w, (((1,), (1,)), ((), ())), preferred_element_type=jnp.float32
    )
    logits = logits + b_ref[...]
    m = jnp.max(logits, axis=-1, keepdims=True)
    e = jnp.exp(logits - m)
    s = jnp.sum(e, axis=-1, keepdims=True)
    scores = e * (1.0 / s)
    scores_ref[...] = scores

    iota = lax.broadcasted_iota(jnp.int32, scores.shape, 1)
    m1 = jnp.max(scores, axis=-1, keepdims=True)
    i1 = jnp.min(jnp.where(scores == m1, iota, _E), axis=-1, keepdims=True)
    masked = jnp.where(iota == i1, -jnp.inf, scores)
    m2 = jnp.max(masked, axis=-1, keepdims=True)
    i2 = jnp.min(jnp.where(masked == m2, iota, _E), axis=-1, keepdims=True)
    ts_ref[...] = jnp.concatenate([m1, m2], axis=-1)
    ti_ref[...] = jnp.concatenate([i1, i2], axis=-1)


def kernel(x, W, b):
    scores, ts, ti = pl.pallas_call(
        _router_body,
        grid=(_N_TOKENS // _TM,),
        in_specs=[
            pl.BlockSpec((_TM, _D), lambda i: (i, 0)),
            pl.BlockSpec((_E, _D), lambda i: (0, 0)),
            pl.BlockSpec((1, _E), lambda i: (0, 0)),
        ],
        out_specs=[
            pl.BlockSpec((_TM, _E), lambda i: (i, 0)),
            pl.BlockSpec((_TM, 2), lambda i: (i, 0)),
            pl.BlockSpec((_TM, 2), lambda i: (i, 0)),
        ],
        out_shape=[
            jax.ShapeDtypeStruct((_N_TOKENS, _E), jnp.float32),
            jax.ShapeDtypeStruct((_N_TOKENS, 2), jnp.float32),
            jax.ShapeDtypeStruct((_N_TOKENS, 2), jnp.int32),
        ],
    )(x, W, b.reshape(1, _E))
    return ts, ti, scores


# speedup vs baseline: 2.0962x; 1.0462x over previous
import jax
import jax.numpy as jnp
from jax import lax
from jax.experimental import pallas as pl

_N_TOKENS = 32768
_D = 768
_E = 64
_TM = 4096


def _router_body(x_ref, w_ref, b_ref, scores_ref, ts_ref, ti_ref):
    x = x_ref[...]
    w = w_ref[...]
    logits = lax.dot_general(
        x, w, (((1,), (1,)), ((), ())), preferred_element_type=jnp.float32
    )
    logits = logits + b_ref[...]
    m = jnp.max(logits, axis=-1, keepdims=True)
    e = jnp.exp(logits - m)
    s = jnp.sum(e, axis=-1, keepdims=True)
    scores = e * (1.0 / s)
    scores_ref[...] = scores

    iota = lax.broadcasted_iota(jnp.int32, scores.shape, 1)
    m1 = jnp.max(scores, axis=-1, keepdims=True)
    i1 = jnp.min(jnp.where(scores == m1, iota, _E), axis=-1, keepdims=True)
    masked = jnp.where(iota == i1, -jnp.inf, scores)
    m2 = jnp.max(masked, axis=-1, keepdims=True)
    i2 = jnp.min(jnp.where(masked == m2, iota, _E), axis=-1, keepdims=True)
    ts_ref[...] = jnp.concatenate([m1, m2], axis=-1)
    ti_ref[...] = jnp.concatenate([i1, i2], axis=-1)


def kernel(x, W, b):
    scores, ts, ti = pl.pallas_call(
        _router_body,
        grid=(_N_TOKENS // _TM,),
        in_specs=[
            pl.BlockSpec((_TM, _D), lambda i: (i, 0)),
            pl.BlockSpec((_E, _D), lambda i: (0, 0)),
            pl.BlockSpec((1, _E), lambda i: (0, 0)),
        ],
        out_specs=[
            pl.BlockSpec((_TM, _E), lambda i: (i, 0)),
            pl.BlockSpec((_TM, 2), lambda i: (i, 0)),
            pl.BlockSpec((_TM, 2), lambda i: (i, 0)),
        ],
        out_shape=[
            jax.ShapeDtypeStruct((_N_TOKENS, _E), jnp.float32),
            jax.ShapeDtypeStruct((_N_TOKENS, 2), jnp.float32),
            jax.ShapeDtypeStruct((_N_TOKENS, 2), jnp.int32),
        ],
    )(x, W, b.reshape(1, _E))
    return ts, ti, scores


# speedup vs baseline: 2.1032x; 1.0033x over previous
import jax
import jax.numpy as jnp
from jax import lax
from jax.experimental import pallas as pl

_N_TOKENS = 32768
_D = 768
_E = 64
_TM = 4096


def _router_body(x_ref, w_ref, b_ref, scores_ref, ts_ref, ti_ref):
    x = x_ref[...]
    w = w_ref[...]
    logits = lax.dot_general(
        x, w, (((1,), (1,)), ((), ())), preferred_element_type=jnp.float32
    )
    logits = logits + b_ref[...]
    m = jnp.max(logits, axis=-1, keepdims=True)
    e = jnp.exp(logits - m)
    s = jnp.sum(e, axis=-1, keepdims=True)
    scores = e * (1.0 / s)
    scores_ref[...] = scores

    bits = lax.bitcast_convert_type(scores, jnp.int32)
    iota = lax.broadcasted_iota(jnp.int32, scores.shape, 1)
    key = (bits & jnp.int32(~63)) | (jnp.int32(63) - iota)
    k1 = jnp.max(key, axis=-1, keepdims=True)
    k2 = jnp.max(jnp.where(key == k1, jnp.int32(0), key), axis=-1,
                 keepdims=True)
    k12 = jnp.concatenate([k1, k2], axis=-1)
    ts_ref[...] = lax.bitcast_convert_type(k12 & jnp.int32(~63), jnp.float32)
    ti_ref[...] = jnp.int32(63) - (k12 & jnp.int32(63))


def kernel(x, W, b):
    scores, ts, ti = pl.pallas_call(
        _router_body,
        grid=(_N_TOKENS // _TM,),
        in_specs=[
            pl.BlockSpec((_TM, _D), lambda i: (i, 0)),
            pl.BlockSpec((_E, _D), lambda i: (0, 0)),
            pl.BlockSpec((1, _E), lambda i: (0, 0)),
        ],
        out_specs=[
            pl.BlockSpec((_TM, _E), lambda i: (i, 0)),
            pl.BlockSpec((_TM, 2), lambda i: (i, 0)),
            pl.BlockSpec((_TM, 2), lambda i: (i, 0)),
        ],
        out_shape=[
            jax.ShapeDtypeStruct((_N_TOKENS, _E), jnp.float32),
            jax.ShapeDtypeStruct((_N_TOKENS, 2), jnp.float32),
            jax.ShapeDtypeStruct((_N_TOKENS, 2), jnp.int32),
        ],
    )(x, W, b.reshape(1, _E))
    return ts, ti, scores
